# baseline (speedup 1.0000x reference)
import functools

import jax
import jax.numpy as jnp
from jax import lax
from jax.experimental import pallas as pl
from jax.experimental.pallas import tpu as pltpu
from jax.experimental.pallas import tpu_sc as plsc

_N_VOCAB = 1000000
_EMBED_DIM = 64
_BATCH = 16384

_NC = 2
_NS = 16
_NW = _NC * _NS
_BPW = _BATCH // _NW
_CHUNK = 128
_NCHUNK = _BPW // _CHUNK


def _gather_kernel(idx_hbm, w_hbm, b_hbm, w_out, b_out,
                   idx_v, rows_v, brows_v, sem):
    wid = lax.axis_index("s") * _NC + lax.axis_index("c")
    base = wid * _BPW
    pltpu.sync_copy(idx_hbm.at[pl.ds(base, _BPW)], idx_v)
    copies = []
    for j in range(_NCHUNK):
        sl = pl.ds(j * _CHUNK, _CHUNK)
        copies.append(pltpu.async_copy(w_hbm.at[idx_v.at[sl]], rows_v.at[sl], sem))
        copies.append(pltpu.async_copy(b_hbm.at[idx_v.at[sl]], brows_v.at[sl], sem))
    for c in copies:
        c.wait()
    pltpu.sync_copy(rows_v, w_out.at[pl.ds(base, _BPW)])
    pltpu.sync_copy(brows_v, b_out.at[pl.ds(base, _BPW)])


def _run(idx, W, b_flat):
    mesh = plsc.VectorSubcoreMesh(core_axis_name="c", subcore_axis_name="s")
    run = functools.partial(
        pl.kernel,
        mesh=mesh,
        out_type=(
            jax.ShapeDtypeStruct((_BATCH, _EMBED_DIM), jnp.float32),
            jax.ShapeDtypeStruct((_BATCH,), jnp.float32),
        ),
        scratch_types=[
            pltpu.VMEM((_BPW,), jnp.int32),
            pltpu.VMEM((_BPW, _EMBED_DIM), jnp.float32),
            pltpu.VMEM((_BPW,), jnp.float32),
            pltpu.SemaphoreType.DMA,
        ],
        compiler_params=pltpu.CompilerParams(use_tc_tiling_on_sc=False),
    )(_gather_kernel)
    return run(idx, W, b_flat)


@jax.jit
def kernel(idx, W, b):
    idx = idx.astype(jnp.int32)
    w_out, b_out = _run(idx, W, jnp.sum(b, axis=1))
    return w_out, b_out.reshape(_BATCH, 1)


# speedup vs baseline: 1.5735x; 1.5735x over previous
import functools

import jax
import jax.numpy as jnp
from jax import lax
from jax.experimental import pallas as pl
from jax.experimental.pallas import tpu as pltpu
from jax.experimental.pallas import tpu_sc as plsc

_N_VOCAB = 1000000
_EMBED_DIM = 64
_BATCH = 16384

_NC = 2
_NS = 16
_NW = _NC * _NS
_BPW = _BATCH // _NW
_CH = 64
_NCH = _BPW // _CH
_BCHUNK = 128


def _w_kernel(idx_hbm, w_hbm, w_out, idx_v, rows_v, sem):
    wid = lax.axis_index("s") * _NC + lax.axis_index("c")
    base = wid * _BPW
    pltpu.sync_copy(idx_hbm.at[pl.ds(base, _BPW)], idx_v)

    for c in range(_NCH):
        def fire(g, _):
            off = pl.multiple_of(c * _CH + g * 16, 16)
            grp = idx_v[pl.ds(off, 16)]
            for k in range(16):
                r = lax.squeeze(lax.slice(grp, (k,), (k + 1,)), (0,))
                pltpu.async_copy(
                    w_hbm.at[pl.ds(r, 1)], rows_v.at[pl.ds(off + k, 1)], sem)
            return 0
        lax.fori_loop(0, _CH // 16, fire, 0)

        def drain(k, _):
            slot = c * _CH + k
            pltpu.make_async_copy(
                w_hbm.at[pl.ds(0, 1)], rows_v.at[pl.ds(slot, 1)], sem).wait()
            return 0
        lax.fori_loop(0, _CH, drain, 0)

    pltpu.sync_copy(rows_v, w_out.at[pl.ds(base, _BPW)])


def _b_kernel(idx_hbm, b_hbm, b_out, idx_v, bvals_v, sem):
    wid = lax.axis_index("s") * _NC + lax.axis_index("c")
    base = wid * _BPW
    pltpu.sync_copy(idx_hbm.at[pl.ds(base, _BPW)], idx_v)
    copies = []
    for j in range(_BPW // _BCHUNK):
        sl = pl.ds(j * _BCHUNK, _BCHUNK)
        copies.append(pltpu.async_copy(b_hbm.at[idx_v.at[sl]], bvals_v.at[sl], sem))
    for c in copies:
        c.wait()
    pltpu.sync_copy(bvals_v, b_out.at[pl.ds(base, _BPW)])


def _run_w(idx, W):
    mesh = plsc.VectorSubcoreMesh(core_axis_name="c", subcore_axis_name="s")
    run = functools.partial(
        pl.kernel,
        mesh=mesh,
        out_type=jax.ShapeDtypeStruct((_BATCH, _EMBED_DIM), jnp.float32),
        scratch_types=[
            pltpu.VMEM((_BPW,), jnp.int32),
            pltpu.VMEM((_BPW, _EMBED_DIM), jnp.float32),
            pltpu.SemaphoreType.DMA,
        ],
    )(_w_kernel)
    return run(idx, W)


def _run_b(idx, b_flat):
    mesh = plsc.VectorSubcoreMesh(core_axis_name="c", subcore_axis_name="s")
    run = functools.partial(
        pl.kernel,
        mesh=mesh,
        out_type=jax.ShapeDtypeStruct((_BATCH,), jnp.float32),
        scratch_types=[
            pltpu.VMEM((_BPW,), jnp.int32),
            pltpu.VMEM((_BPW,), jnp.float32),
            pltpu.SemaphoreType.DMA,
        ],
        compiler_params=pltpu.CompilerParams(use_tc_tiling_on_sc=False),
    )(_b_kernel)
    return run(idx, b_flat)


@jax.jit
def kernel(idx, W, b):
    idx = idx.astype(jnp.int32)
    w_out = _run_w(idx, W)
    b_out = _run_b(idx, jnp.sum(b, axis=1))
    return w_out, b_out.reshape(_BATCH, 1)


# speedup vs baseline: 1.5758x; 1.0014x over previous
import functools

import jax
import jax.numpy as jnp
from jax import lax
from jax.experimental import pallas as pl
from jax.experimental.pallas import tpu as pltpu
from jax.experimental.pallas import tpu_sc as plsc

_N_VOCAB = 1000000
_EMBED_DIM = 64
_BATCH = 16384

_NC = 2
_NS = 16
_NW = _NC * _NS
_BPW = _BATCH // _NW
_CH = 64
_NCH = _BPW // _CH
_BCHUNK = 128


def _w_kernel(idx_hbm, w_hbm, w_out, idx_v, rows_v, sem):
    wid = lax.axis_index("s") * _NC + lax.axis_index("c")
    base = wid * _BPW
    pltpu.sync_copy(idx_hbm.at[pl.ds(base, _BPW)], idx_v)

    for c in range(_NCH):
        def fire(g, _):
            off = pl.multiple_of(c * _CH + g * 16, 16)
            grp = idx_v[pl.ds(off, 16)]
            for k in range(16):
                r = lax.squeeze(lax.slice(grp, (k,), (k + 1,)), (0,))
                pltpu.async_copy(
                    w_hbm.at[pl.ds(r, 1)], rows_v.at[pl.ds(off + k, 1)], sem)
            return 0
        lax.fori_loop(0, _CH // 16, fire, 0)

        def drain(k, _):
            slot = c * _CH + k
            pltpu.make_async_copy(
                w_hbm.at[pl.ds(0, 1)], rows_v.at[pl.ds(slot, 1)], sem).wait()
            return 0
        lax.fori_loop(0, _CH, drain, 0)

    pltpu.sync_copy(rows_v, w_out.at[pl.ds(base, _BPW)])


def _b_kernel(idx_hbm, b_hbm, b_out, idx_v, bvals_v, sem):
    wid = lax.axis_index("s") * _NC + lax.axis_index("c")
    base = wid * _BPW
    pltpu.sync_copy(idx_hbm.at[pl.ds(base, _BPW)], idx_v)
    copies = []
    for j in range(_BPW // _BCHUNK):
        sl = pl.ds(j * _BCHUNK, _BCHUNK)
        copies.append(pltpu.async_copy(b_hbm.at[idx_v.at[sl]], bvals_v.at[sl], sem))
    for c in copies:
        c.wait()
    pltpu.sync_copy(bvals_v, b_out.at[pl.ds(base, _BPW)])


def _run_w(idx, W):
    mesh = plsc.VectorSubcoreMesh(core_axis_name="c", subcore_axis_name="s")
    run = functools.partial(
        pl.kernel,
        mesh=mesh,
        out_type=jax.ShapeDtypeStruct((_BATCH, _EMBED_DIM), jnp.float32),
        scratch_types=[
            pltpu.VMEM((_BPW,), jnp.int32),
            pltpu.VMEM((_BPW, _EMBED_DIM), jnp.float32),
            pltpu.SemaphoreType.DMA,
        ],
    )(_w_kernel)
    return run(idx, W)


def _run_b(idx, b_flat):
    mesh = plsc.VectorSubcoreMesh(core_axis_name="c", subcore_axis_name="s")
    run = functools.partial(
        pl.kernel,
        mesh=mesh,
        out_type=jax.ShapeDtypeStruct((_BATCH,), jnp.float32),
        scratch_types=[
            pltpu.VMEM((_BPW,), jnp.int32),
            pltpu.VMEM((_BPW,), jnp.float32),
            pltpu.SemaphoreType.DMA,
        ],
        compiler_params=pltpu.CompilerParams(use_tc_tiling_on_sc=False),
    )(_b_kernel)
    return run(idx, b_flat)


@jax.jit
def kernel(idx, W, b):
    idx = idx.astype(jnp.int32)
    b_out = _run_b(idx, jnp.sum(b, axis=1))
    w_out = _run_w(idx, W)
    return w_out, b_out.reshape(_BATCH, 1)
